# Initial kernel scaffold; baseline (speedup 1.0000x reference)
import jax
import jax.numpy as jnp
from jax.experimental import pallas as pl


def kernel(x, edge_index, edge_attr, batch, params):
    raise NotImplementedError("write your pallas kernel here")


# speedup vs baseline: 3.0302x; 3.0302x over previous
import functools

import jax
import jax.numpy as jnp
from jax import lax
from jax.experimental import pallas as pl
from jax.experimental.pallas import tpu as pltpu
from jax.experimental.pallas import tpu_sc as plsc

N = 10000
E = 320000
F = 128
NC = 2
NS = 16
EPT = E // (NC * NS)
K = 80
NWIN = EPT // K
NCHUNK = N // K

_HIGHEST = jax.lax.Precision.HIGHEST


def _dot(a, b):
    return jax.lax.dot_general(a, b, (((1,), (0,)), ((), ())),
                               precision=_HIGHEST,
                               preferred_element_type=jnp.float32)


def _sc_msg_body(h_hbm, e_hbm, src_hbm, dst_hbm, out_hbm,
                 aggr_sh, src_v, dst_v, e_v, rows_v, sem):
    c = lax.axis_index("c")
    s = lax.axis_index("s")

    def zbody(i, _):
        for j in range(F // 16):
            rows_v[i, pl.ds(j * 16, 16)] = jnp.zeros((16,), jnp.float32)
        return 0
    lax.fori_loop(0, K, zbody, 0)

    def zchunk(q, _):
        ch = q * NS + s

        @pl.when(ch < NCHUNK)
        def _():
            pltpu.sync_copy(rows_v, aggr_sh.at[pl.ds(ch * K, K)])
        return 0
    lax.fori_loop(0, (NCHUNK + NS - 1) // NS, zchunk, 0)
    plsc.subcore_barrier()

    ebase = (c * NS + s) * EPT

    def win(w, _):
        base = ebase + w * K
        pltpu.sync_copy(src_hbm.at[pl.ds(base, K)], src_v)
        pltpu.sync_copy(dst_hbm.at[pl.ds(base, K)], dst_v)
        gcp = pltpu.async_copy(h_hbm.at[src_v], rows_v, sem)
        pltpu.sync_copy(e_hbm.at[pl.ds(base, K)], e_v)
        gcp.wait()

        def cbody(i, _):
            for j in range(F // 16):
                v = rows_v[i, pl.ds(j * 16, 16)] + e_v[i, pl.ds(j * 16, 16)]
                rows_v[i, pl.ds(j * 16, 16)] = jnp.maximum(v, 0.0)
            return 0
        lax.fori_loop(0, K, cbody, 0)
        pltpu.sync_copy(rows_v, aggr_sh.at[dst_v], add=True)
        return 0

    lax.fori_loop(0, NWIN, win, 0)
    plsc.subcore_barrier()

    def wchunk(q, _):
        ch = q * NS + s

        @pl.when(ch < NCHUNK)
        def _():
            pltpu.sync_copy(aggr_sh.at[pl.ds(ch * K, K)],
                            out_hbm.at[c].at[pl.ds(ch * K, K)])
        return 0
    lax.fori_loop(0, (NCHUNK + NS - 1) // NS, wchunk, 0)


@functools.cache
def _get_sc_msg():
    return pl.kernel(
        _sc_msg_body,
        out_type=jax.ShapeDtypeStruct((NC, N, F), jnp.float32),
        mesh=plsc.VectorSubcoreMesh(core_axis_name="c", subcore_axis_name="s",
                                    num_cores=NC, num_subcores=NS),
        scratch_types=[
            pltpu.VMEM_SHARED((N, F), jnp.float32),
            pltpu.VMEM((K,), jnp.int32),
            pltpu.VMEM((K,), jnp.int32),
            pltpu.VMEM((K, F), jnp.float32),
            pltpu.VMEM((K, F), jnp.float32),
            pltpu.SemaphoreType.DMA,
        ],
    )


def _sc_msg(h, e, src, dst):
    return _get_sc_msg()(h, e, src, dst)


def _lin_body(x_ref, w_ref, b_ref, o_ref):
    o_ref[...] = _dot(x_ref[...], w_ref[...]) + b_ref[...]


def _linear(x, w, b, br):
    r, fi = x.shape
    fo = w.shape[1]
    grid = r // br
    return pl.pallas_call(
        _lin_body,
        grid=(grid,),
        in_specs=[
            pl.BlockSpec((br, fi), lambda i: (i, 0)),
            pl.BlockSpec((fi, fo), lambda i: (0, 0)),
            pl.BlockSpec((1, fo), lambda i: (0, 0)),
        ],
        out_specs=pl.BlockSpec((br, fo), lambda i: (i, 0)),
        out_shape=jax.ShapeDtypeStruct((r, fo), jnp.float32),
    )(x, w, b.reshape(1, fo))


def _mm_stats_body(h_ref, a_ref, eps_ref, w_ref, b_ref, z_ref, st_ref):
    i = pl.program_id(0)
    hb = (1.0 + eps_ref[0, 0]) * h_ref[...] + a_ref[0] + a_ref[1]
    z = _dot(hb, w_ref[...]) + b_ref[...]
    z_ref[...] = z

    @pl.when(i == 0)
    def _():
        st_ref[...] = jnp.zeros_like(st_ref)
    st_ref[0, :] += jnp.sum(z, axis=0)
    st_ref[1, :] += jnp.sum(z * z, axis=0)


def _mm_stats(h, a2, eps, w, b, br):
    r, fi = h.shape
    fo = w.shape[1]
    grid = r // br
    return pl.pallas_call(
        _mm_stats_body,
        grid=(grid,),
        in_specs=[
            pl.BlockSpec((br, fi), lambda i: (i, 0)),
            pl.BlockSpec((2, br, fi), lambda i: (0, i, 0)),
            pl.BlockSpec(memory_space=pltpu.SMEM),
            pl.BlockSpec((fi, fo), lambda i: (0, 0)),
            pl.BlockSpec((1, fo), lambda i: (0, 0)),
        ],
        out_specs=[
            pl.BlockSpec((br, fo), lambda i: (i, 0)),
            pl.BlockSpec((2, fo), lambda i: (0, 0)),
        ],
        out_shape=[
            jax.ShapeDtypeStruct((r, fo), jnp.float32),
            jax.ShapeDtypeStruct((2, fo), jnp.float32),
        ],
    )(h, a2, eps.reshape(1, 1), w, b.reshape(1, fo))


def _bn_mm_stats_body(z_ref, st_ref, g_ref, bb_ref, w_ref, b_ref,
                      y_ref, st2_ref):
    i = pl.program_id(0)
    inv_n = 1.0 / N
    mu = st_ref[0:1, :] * inv_n
    var = st_ref[1:2, :] * inv_n - mu * mu
    scale = g_ref[...] * jax.lax.rsqrt(var + 1e-5)
    zn = jnp.maximum((z_ref[...] - mu) * scale + bb_ref[...], 0.0)
    y = _dot(zn, w_ref[...]) + b_ref[...]
    y_ref[...] = y

    @pl.when(i == 0)
    def _():
        st2_ref[...] = jnp.zeros_like(st2_ref)
    st2_ref[0, :] += jnp.sum(y, axis=0)
    st2_ref[1, :] += jnp.sum(y * y, axis=0)


def _bn_mm_stats(z, st, g, bb, w, b, br):
    r, fi = z.shape
    fo = w.shape[1]
    grid = r // br
    return pl.pallas_call(
        _bn_mm_stats_body,
        grid=(grid,),
        in_specs=[
            pl.BlockSpec((br, fi), lambda i: (i, 0)),
            pl.BlockSpec((2, fi), lambda i: (0, 0)),
            pl.BlockSpec((1, fi), lambda i: (0, 0)),
            pl.BlockSpec((1, fi), lambda i: (0, 0)),
            pl.BlockSpec((fi, fo), lambda i: (0, 0)),
            pl.BlockSpec((1, fo), lambda i: (0, 0)),
        ],
        out_specs=[
            pl.BlockSpec((br, fo), lambda i: (i, 0)),
            pl.BlockSpec((2, fo), lambda i: (0, 0)),
        ],
        out_shape=[
            jax.ShapeDtypeStruct((r, fo), jnp.float32),
            jax.ShapeDtypeStruct((2, fo), jnp.float32),
        ],
    )(z, st, g.reshape(1, fi), bb.reshape(1, fi), w, b.reshape(1, fo))


def _bn_relu_body(y_ref, st_ref, g_ref, bb_ref, h_ref):
    inv_n = 1.0 / N
    mu = st_ref[0:1, :] * inv_n
    var = st_ref[1:2, :] * inv_n - mu * mu
    scale = g_ref[...] * jax.lax.rsqrt(var + 1e-5)
    h_ref[...] = jnp.maximum((y_ref[...] - mu) * scale + bb_ref[...], 0.0)


def _bn_relu(y, st, g, bb, br):
    r, f = y.shape
    grid = r // br
    return pl.pallas_call(
        _bn_relu_body,
        grid=(grid,),
        in_specs=[
            pl.BlockSpec((br, f), lambda i: (i, 0)),
            pl.BlockSpec((2, f), lambda i: (0, 0)),
            pl.BlockSpec((1, f), lambda i: (0, 0)),
            pl.BlockSpec((1, f), lambda i: (0, 0)),
        ],
        out_specs=pl.BlockSpec((br, f), lambda i: (i, 0)),
        out_shape=jax.ShapeDtypeStruct((r, f), jnp.float32),
    )(y, st, g.reshape(1, f), bb.reshape(1, f))


def _pool_body(h_ref, b_ref, w1_ref, b1_ref, w2_ref, b2_ref, o_ref):
    seg = (jax.lax.broadcasted_iota(jnp.int32, (64, N), 0)
           == b_ref[...]).astype(jnp.float32)
    sums = _dot(seg, h_ref[...])
    counts = jnp.sum(seg, axis=1, keepdims=True)
    pooled = sums / jnp.maximum(counts, 1.0)
    hidden = jnp.maximum(_dot(pooled, w1_ref[...]) + b1_ref[...], 0.0)
    o_ref[...] = _dot(hidden, w2_ref[...]) + b2_ref[...]


def _pool(h, batch, w1, b1, w2, b2):
    f = h.shape[1]
    return pl.pallas_call(
        _pool_body,
        out_shape=jax.ShapeDtypeStruct((64, f), jnp.float32),
    )(h, batch.reshape(1, N), w1, b1.reshape(1, f), w2, b2.reshape(1, f))


def kernel(x, edge_index, edge_attr, batch, params):
    src = edge_index[0]
    dst = edge_index[1]
    h = _linear(x, params['node_w'], params['node_b'], 2000)
    e = _linear(edge_attr, params['edge_w'], params['edge_b'], 8000)
    for lp in params['layers']:
        a2 = _sc_msg(h, e, src, dst)
        z, st1 = _mm_stats(h, a2, lp['eps'], lp['w1'], lp['b1'], 2000)
        y, st2 = _bn_mm_stats(z, st1, lp['bn1_g'], lp['bn1_b'],
                              lp['w2'], lp['b2'], 2000)
        h = _bn_relu(y, st2, lp['bn_g'], lp['bn_b'], 2000)
    return _pool(h, batch, params['rw1'], params['rb1'],
                 params['rw2'], params['rb2'])


# speedup vs baseline: 3.4681x; 1.1445x over previous
import functools

import jax
import jax.numpy as jnp
from jax import lax
from jax.experimental import pallas as pl
from jax.experimental.pallas import tpu as pltpu
from jax.experimental.pallas import tpu_sc as plsc

N = 10000
E = 320000
F = 128
FH = 64
NC = 2
NS = 16
K2 = 64
NWIN = 320
EPT = NWIN * K2
E2 = NS * EPT
NB = 4
NEB = 2
CW = 160
NCHK = NWIN // CW
NPADROW = 112
NA = N + NPADROW

_HIGHEST = jax.lax.Precision.HIGHEST


def _dot(a, b):
    return jax.lax.dot_general(a, b, (((1,), (0,)), ((), ())),
                               precision=_HIGHEST,
                               preferred_element_type=jnp.float32)


def _sc_msg_body(h2, e2, src4, dst4, out,
                 table_sh, accum_sh, srcv, dstv, rows, ebuf,
                 gsem, esem, ssem, csem):
    c = lax.axis_index("c")
    s = lax.axis_index("s")

    def zb(i, _):
        for j in range(FH // 16):
            rows[0, i, pl.ds(j * 16, 16)] = jnp.zeros((16,), jnp.float32)
        return 0
    lax.fori_loop(0, K2, zb, 0)

    for q in range(8):
        ch = q * NS + s

        @pl.when(ch < 125)
        def _():
            pltpu.async_copy(h2.at[c].at[pl.ds(ch * 80, 80)],
                             table_sh.at[pl.ds(ch * 80, 80)], csem)
    for q in range(10):
        ch = q * NS + s

        @pl.when(ch < NA // K2)
        def _():
            pltpu.async_copy(rows.at[0],
                             accum_sh.at[pl.ds(ch * K2, K2)], csem)

    for q in range(8):
        ch = q * NS + s

        @pl.when(ch < 125)
        def _():
            pltpu.make_async_copy(h2.at[c].at[pl.ds(ch * 80, 80)],
                                  table_sh.at[pl.ds(ch * 80, 80)],
                                  csem).wait()
    for q in range(10):
        ch = q * NS + s

        @pl.when(ch < NA // K2)
        def _():
            pltpu.make_async_copy(rows.at[0],
                                  accum_sh.at[pl.ds(ch * K2, K2)],
                                  csem).wait()

    plsc.subcore_barrier()

    ebase = s * EPT

    for chunk in range(NCHK):
        cbase = ebase + chunk * CW * K2
        pltpu.sync_copy(src4.at[s].at[pl.ds(chunk * CW, CW)], srcv)
        pltpu.sync_copy(dst4.at[s].at[pl.ds(chunk * CW, CW)], dstv)

        for b in range(2):
            pltpu.async_copy(table_sh.at[srcv.at[b]], rows.at[b], gsem.at[b])
            pltpu.async_copy(e2.at[c].at[pl.ds(cbase + b * K2, K2)],
                             ebuf.at[b], esem.at[b])

        def qstep(q, _):
            for b in range(NB):
                lw = q * NB + b
                be = b % NEB
                pltpu.make_async_copy(table_sh.at[srcv.at[lw]], rows.at[b],
                                      gsem.at[b]).wait()
                pltpu.make_async_copy(
                    e2.at[c].at[pl.ds(cbase + lw * K2, K2)],
                    ebuf.at[be], esem.at[be]).wait()

                def cb(i, _):
                    for j in range(FH // 16):
                        sl = pl.ds(j * 16, 16)
                        rows[b, i, sl] = jnp.maximum(
                            rows[b, i, sl] + ebuf[be, i, sl], 0.0)
                    return 0
                lax.fori_loop(0, K2, cb, 0)

                @pl.when(lw + 2 < CW)
                def _():
                    pltpu.async_copy(
                        e2.at[c].at[pl.ds(cbase + (lw + 2) * K2, K2)],
                        ebuf.at[be], esem.at[be])

                pltpu.async_copy(rows.at[b], accum_sh.at[dstv.at[lw]],
                                 ssem.at[b], add=True)

                b2 = (b + 2) % NB

                @pl.when(lw + 2 < CW)
                def _():
                    @pl.when(lw >= 2)
                    def _():
                        pltpu.make_async_copy(rows.at[b2],
                                              accum_sh.at[dstv.at[lw - 2]],
                                              ssem.at[b2]).wait()
                    pltpu.async_copy(table_sh.at[srcv.at[lw + 2]],
                                     rows.at[b2], gsem.at[b2])
            return 0
        lax.fori_loop(0, CW // NB, qstep, 0)

        for b in range(NB):
            lw = CW - NB + b
            pltpu.make_async_copy(rows.at[b], accum_sh.at[dstv.at[lw]],
                                  ssem.at[b]).wait()

    plsc.subcore_barrier()

    for q in range(8):
        ch = q * NS + s

        @pl.when(ch < 125)
        def _():
            pltpu.async_copy(accum_sh.at[pl.ds(ch * 80, 80)],
                             out.at[c].at[pl.ds(ch * 80, 80)], csem)
    for q in range(8):
        ch = q * NS + s

        @pl.when(ch < 125)
        def _():
            pltpu.make_async_copy(accum_sh.at[pl.ds(ch * 80, 80)],
                                  out.at[c].at[pl.ds(ch * 80, 80)],
                                  csem).wait()


@functools.cache
def _get_sc_msg():
    return pl.kernel(
        _sc_msg_body,
        out_type=jax.ShapeDtypeStruct((NC, N, FH), jnp.float32),
        mesh=plsc.VectorSubcoreMesh(core_axis_name="c", subcore_axis_name="s",
                                    num_cores=NC, num_subcores=NS),
        compiler_params=pltpu.CompilerParams(use_tc_tiling_on_sc=False),
        scratch_types=[
            pltpu.VMEM_SHARED((N, FH), jnp.float32),
            pltpu.VMEM_SHARED((NA, FH), jnp.float32),
            pltpu.VMEM((CW, K2), jnp.int32),
            pltpu.VMEM((CW, K2), jnp.int32),
            pltpu.VMEM((NB, K2, FH), jnp.float32),
            pltpu.VMEM((NEB, K2, FH), jnp.float32),
            pltpu.SemaphoreType.DMA((NB,)),
            pltpu.SemaphoreType.DMA((NEB,)),
            pltpu.SemaphoreType.DMA((NB,)),
            pltpu.SemaphoreType.DMA,
        ],
    )


def _sc_msg(h2, e2, src4, dst4):
    return _get_sc_msg()(h2, e2, src4, dst4)


def _lin2_body(x_ref, w_ref, b_ref, o_ref):
    o_ref[0] = _dot(x_ref[...], w_ref[0]) + b_ref[0]
    o_ref[1] = _dot(x_ref[...], w_ref[1]) + b_ref[1]


def _linear2(x, w2, b2, br):
    r, fi = x.shape
    grid = r // br
    return pl.pallas_call(
        _lin2_body,
        grid=(grid,),
        in_specs=[
            pl.BlockSpec((br, fi), lambda i: (i, 0)),
            pl.BlockSpec((2, fi, FH), lambda i: (0, 0, 0)),
            pl.BlockSpec((2, 1, FH), lambda i: (0, 0, 0)),
        ],
        out_specs=pl.BlockSpec((2, br, FH), lambda i: (0, i, 0)),
        out_shape=jax.ShapeDtypeStruct((2, r, FH), jnp.float32),
    )(x, w2, b2)


def _mm_stats_body(h_ref, a_ref, eps_ref, w_ref, b_ref, z_ref, st_ref):
    i = pl.program_id(0)
    e1 = 1.0 + eps_ref[0, 0]
    zin0 = e1 * h_ref[0] + a_ref[0]
    zin1 = e1 * h_ref[1] + a_ref[1]
    z = (_dot(zin0, w_ref[0:FH, :]) + _dot(zin1, w_ref[FH:2 * FH, :])
         + b_ref[...])
    z_ref[...] = z

    @pl.when(i == 0)
    def _():
        st_ref[...] = jnp.zeros_like(st_ref)
    st_ref[0, :] += jnp.sum(z, axis=0)
    st_ref[1, :] += jnp.sum(z * z, axis=0)


def _mm_stats(h2, a2, eps, w, b, br):
    fo = w.shape[1]
    grid = N // br
    return pl.pallas_call(
        _mm_stats_body,
        grid=(grid,),
        in_specs=[
            pl.BlockSpec((2, br, FH), lambda i: (0, i, 0)),
            pl.BlockSpec((2, br, FH), lambda i: (0, i, 0)),
            pl.BlockSpec(memory_space=pltpu.SMEM),
            pl.BlockSpec((F, fo), lambda i: (0, 0)),
            pl.BlockSpec((1, fo), lambda i: (0, 0)),
        ],
        out_specs=[
            pl.BlockSpec((br, fo), lambda i: (i, 0)),
            pl.BlockSpec((2, fo), lambda i: (0, 0)),
        ],
        out_shape=[
            jax.ShapeDtypeStruct((N, fo), jnp.float32),
            jax.ShapeDtypeStruct((2, fo), jnp.float32),
        ],
    )(h2, a2, eps.reshape(1, 1), w, b.reshape(1, fo))


def _bn_mm_stats_body(z_ref, st_ref, g_ref, bb_ref, w_ref, b_ref,
                      y_ref, st2_ref):
    i = pl.program_id(0)
    inv_n = 1.0 / N
    mu = st_ref[0:1, :] * inv_n
    var = st_ref[1:2, :] * inv_n - mu * mu
    scale = g_ref[...] * jax.lax.rsqrt(var + 1e-5)
    zn = jnp.maximum((z_ref[...] - mu) * scale + bb_ref[...], 0.0)
    y = _dot(zn, w_ref[...]) + b_ref[...]
    y_ref[...] = y

    @pl.when(i == 0)
    def _():
        st2_ref[...] = jnp.zeros_like(st2_ref)
    st2_ref[0, :] += jnp.sum(y, axis=0)
    st2_ref[1, :] += jnp.sum(y * y, axis=0)


def _bn_mm_stats(z, st, g, bb, w, b, br):
    fi = z.shape[1]
    fo = w.shape[1]
    grid = N // br
    return pl.pallas_call(
        _bn_mm_stats_body,
        grid=(grid,),
        in_specs=[
            pl.BlockSpec((br, fi), lambda i: (i, 0)),
            pl.BlockSpec((2, fi), lambda i: (0, 0)),
            pl.BlockSpec((1, fi), lambda i: (0, 0)),
            pl.BlockSpec((1, fi), lambda i: (0, 0)),
            pl.BlockSpec((fi, fo), lambda i: (0, 0)),
            pl.BlockSpec((1, fo), lambda i: (0, 0)),
        ],
        out_specs=[
            pl.BlockSpec((br, fo), lambda i: (i, 0)),
            pl.BlockSpec((2, fo), lambda i: (0, 0)),
        ],
        out_shape=[
            jax.ShapeDtypeStruct((N, fo), jnp.float32),
            jax.ShapeDtypeStruct((2, fo), jnp.float32),
        ],
    )(z, st, g.reshape(1, fi), bb.reshape(1, fi), w, b.reshape(1, fo))


def _bn_relu2_body(y_ref, st_ref, g_ref, bb_ref, h_ref):
    inv_n = 1.0 / N
    mu = st_ref[0:1, :] * inv_n
    var = st_ref[1:2, :] * inv_n - mu * mu
    scale = g_ref[...] * jax.lax.rsqrt(var + 1e-5)
    full = jnp.maximum((y_ref[...] - mu) * scale + bb_ref[...], 0.0)
    h_ref[0] = full[:, 0:FH]
    h_ref[1] = full[:, FH:2 * FH]


def _bn_relu2(y, st, g, bb, br):
    f = y.shape[1]
    grid = N // br
    return pl.pallas_call(
        _bn_relu2_body,
        grid=(grid,),
        in_specs=[
            pl.BlockSpec((br, f), lambda i: (i, 0)),
            pl.BlockSpec((2, f), lambda i: (0, 0)),
            pl.BlockSpec((1, f), lambda i: (0, 0)),
            pl.BlockSpec((1, f), lambda i: (0, 0)),
        ],
        out_specs=pl.BlockSpec((2, br, FH), lambda i: (0, i, 0)),
        out_shape=jax.ShapeDtypeStruct((2, N, FH), jnp.float32),
    )(y, st, g.reshape(1, f), bb.reshape(1, f))


def _pool_body(h_ref, b_ref, w1_ref, b1_ref, w2_ref, b2_ref, o_ref):
    seg = (jax.lax.broadcasted_iota(jnp.int32, (64, N), 0)
           == b_ref[...]).astype(jnp.float32)
    s0 = _dot(seg, h_ref[0])
    s1 = _dot(seg, h_ref[1])
    counts = jnp.maximum(jnp.sum(seg, axis=1, keepdims=True), 1.0)
    p0 = s0 / counts
    p1 = s1 / counts
    hidden = jnp.maximum(
        _dot(p0, w1_ref[0:FH, :]) + _dot(p1, w1_ref[FH:2 * FH, :])
        + b1_ref[...], 0.0)
    o_ref[...] = _dot(hidden, w2_ref[...]) + b2_ref[...]


def _pool(h2, batch, w1, b1, w2, b2):
    f = w1.shape[0]
    return pl.pallas_call(
        _pool_body,
        out_shape=jax.ShapeDtypeStruct((64, f), jnp.float32),
    )(h2, batch.reshape(1, N), w1, b1.reshape(1, f), w2, b2.reshape(1, f))


def _split_w(w):
    fi = w.shape[0]
    return w.reshape(fi, 2, FH).transpose(1, 0, 2)


def _split_b(b):
    return b.reshape(2, 1, FH)


def kernel(x, edge_index, edge_attr, batch, params):
    npad = E2 - E
    pidx = jnp.arange(npad, dtype=jnp.int32)
    src = jnp.concatenate([edge_index[0], (pidx * 97) % N])
    dst = jnp.concatenate([edge_index[1], N + (pidx % NPADROW)])
    src4 = src.reshape(NS, NWIN, K2)
    dst4 = dst.reshape(NS, NWIN, K2)
    ea = jnp.pad(edge_attr, ((0, npad), (0, 0)))

    h2 = _linear2(x, _split_w(params['node_w']), _split_b(params['node_b']),
                  2000)
    e2 = _linear2(ea, _split_w(params['edge_w']), _split_b(params['edge_b']),
                  8192)
    for lp in params['layers']:
        a2 = _sc_msg(h2, e2, src4, dst4)
        z, st1 = _mm_stats(h2, a2, lp['eps'], lp['w1'], lp['b1'], 2000)
        y, st2 = _bn_mm_stats(z, st1, lp['bn1_g'], lp['bn1_b'],
                              lp['w2'], lp['b2'], 2000)
        h2 = _bn_relu2(y, st2, lp['bn_g'], lp['bn_b'], 2000)
    return _pool(h2, batch, params['rw1'], params['rb1'],
                 params['rw2'], params['rb2'])


# speedup vs baseline: 3.7592x; 1.0839x over previous
import functools

import jax
import jax.numpy as jnp
from jax import lax
from jax.experimental import pallas as pl
from jax.experimental.pallas import tpu as pltpu
from jax.experimental.pallas import tpu_sc as plsc

N = 10000
E = 320000
F = 128
FH = 64
NC = 2
NS = 16
K2 = 64
NWIN = 320
EPT = NWIN * K2
E2 = NS * EPT
NB = 4
NEB = 2
CW = 160
NCHK = NWIN // CW
NPADROW = 112
NA = N + NPADROW

_HIGHEST = jax.lax.Precision.HIGHEST


def _dot(a, b):
    return jax.lax.dot_general(a, b, (((1,), (0,)), ((), ())),
                               precision=_HIGHEST,
                               preferred_element_type=jnp.float32)


def _sc_msg_body(h2, e2, src4, dst4, out,
                 table_sh, accum_sh, srcv, dstv, rows, ebuf,
                 gsem, esem, ssem, csem):
    c = lax.axis_index("c")
    s = lax.axis_index("s")

    def zb(i, _):
        for j in range(FH // 16):
            rows[0, i, pl.ds(j * 16, 16)] = jnp.zeros((16,), jnp.float32)
        return 0
    lax.fori_loop(0, K2, zb, 0)

    for q in range(8):
        ch = q * NS + s

        @pl.when(ch < 125)
        def _():
            pltpu.async_copy(h2.at[c].at[pl.ds(ch * 80, 80)],
                             table_sh.at[pl.ds(ch * 80, 80)], csem)
    for q in range(10):
        ch = q * NS + s

        @pl.when(ch < NA // K2)
        def _():
            pltpu.async_copy(rows.at[0],
                             accum_sh.at[pl.ds(ch * K2, K2)], csem)

    for q in range(8):
        ch = q * NS + s

        @pl.when(ch < 125)
        def _():
            pltpu.make_async_copy(h2.at[c].at[pl.ds(ch * 80, 80)],
                                  table_sh.at[pl.ds(ch * 80, 80)],
                                  csem).wait()
    for q in range(10):
        ch = q * NS + s

        @pl.when(ch < NA // K2)
        def _():
            pltpu.make_async_copy(rows.at[0],
                                  accum_sh.at[pl.ds(ch * K2, K2)],
                                  csem).wait()

    plsc.subcore_barrier()

    ebase = s * EPT

    for chunk in range(NCHK):
        cbase = ebase + chunk * CW * K2
        pltpu.sync_copy(src4.at[s].at[pl.ds(chunk * CW, CW)], srcv)
        pltpu.sync_copy(dst4.at[s].at[pl.ds(chunk * CW, CW)], dstv)

        for b in range(2):
            pltpu.async_copy(table_sh.at[srcv.at[b]], rows.at[b], gsem.at[b])
            pltpu.async_copy(e2.at[c].at[pl.ds(cbase + b * K2, K2)],
                             ebuf.at[b], esem.at[b])

        def qstep(q, _):
            for b in range(NB):
                lw = q * NB + b
                be = b % NEB
                pltpu.make_async_copy(table_sh.at[srcv.at[lw]], rows.at[b],
                                      gsem.at[b]).wait()
                pltpu.make_async_copy(
                    e2.at[c].at[pl.ds(cbase + lw * K2, K2)],
                    ebuf.at[be], esem.at[be]).wait()

                @functools.partial(plsc.parallel_loop, 0, K2, unroll=8)
                def _(i):
                    for j in range(FH // 16):
                        sl = pl.ds(j * 16, 16)
                        rows[b, i, sl] = jnp.maximum(
                            rows[b, i, sl] + ebuf[be, i, sl], 0.0)

                @pl.when(lw + 2 < CW)
                def _():
                    pltpu.async_copy(
                        e2.at[c].at[pl.ds(cbase + (lw + 2) * K2, K2)],
                        ebuf.at[be], esem.at[be])

                pltpu.async_copy(rows.at[b], accum_sh.at[dstv.at[lw]],
                                 ssem.at[b], add=True)

                b2 = (b + 2) % NB

                @pl.when(lw + 2 < CW)
                def _():
                    @pl.when(lw >= 2)
                    def _():
                        pltpu.make_async_copy(rows.at[b2],
                                              accum_sh.at[dstv.at[lw - 2]],
                                              ssem.at[b2]).wait()
                    pltpu.async_copy(table_sh.at[srcv.at[lw + 2]],
                                     rows.at[b2], gsem.at[b2])
            return 0
        lax.fori_loop(0, CW // NB, qstep, 0)

        for b in range(NB):
            lw = CW - NB + b
            pltpu.make_async_copy(rows.at[b], accum_sh.at[dstv.at[lw]],
                                  ssem.at[b]).wait()

    plsc.subcore_barrier()

    for q in range(8):
        ch = q * NS + s

        @pl.when(ch < 125)
        def _():
            pltpu.async_copy(accum_sh.at[pl.ds(ch * 80, 80)],
                             out.at[c].at[pl.ds(ch * 80, 80)], csem)
    for q in range(8):
        ch = q * NS + s

        @pl.when(ch < 125)
        def _():
            pltpu.make_async_copy(accum_sh.at[pl.ds(ch * 80, 80)],
                                  out.at[c].at[pl.ds(ch * 80, 80)],
                                  csem).wait()


@functools.cache
def _get_sc_msg():
    return pl.kernel(
        _sc_msg_body,
        out_type=jax.ShapeDtypeStruct((NC, N, FH), jnp.float32),
        mesh=plsc.VectorSubcoreMesh(core_axis_name="c", subcore_axis_name="s",
                                    num_cores=NC, num_subcores=NS),
        compiler_params=pltpu.CompilerParams(use_tc_tiling_on_sc=False),
        scratch_types=[
            pltpu.VMEM_SHARED((N, FH), jnp.float32),
            pltpu.VMEM_SHARED((NA, FH), jnp.float32),
            pltpu.VMEM((CW, K2), jnp.int32),
            pltpu.VMEM((CW, K2), jnp.int32),
            pltpu.VMEM((NB, K2, FH), jnp.float32),
            pltpu.VMEM((NEB, K2, FH), jnp.float32),
            pltpu.SemaphoreType.DMA((NB,)),
            pltpu.SemaphoreType.DMA((NEB,)),
            pltpu.SemaphoreType.DMA((NB,)),
            pltpu.SemaphoreType.DMA,
        ],
    )


def _sc_msg(h2, e2, src4, dst4):
    return _get_sc_msg()(h2, e2, src4, dst4)


def _lin2_body(x_ref, w_ref, b_ref, o_ref):
    o_ref[0] = _dot(x_ref[...], w_ref[0]) + b_ref[0]
    o_ref[1] = _dot(x_ref[...], w_ref[1]) + b_ref[1]


def _linear2(x, w2, b2, br):
    r, fi = x.shape
    grid = r // br
    return pl.pallas_call(
        _lin2_body,
        grid=(grid,),
        in_specs=[
            pl.BlockSpec((br, fi), lambda i: (i, 0)),
            pl.BlockSpec((2, fi, FH), lambda i: (0, 0, 0)),
            pl.BlockSpec((2, 1, FH), lambda i: (0, 0, 0)),
        ],
        out_specs=pl.BlockSpec((2, br, FH), lambda i: (0, i, 0)),
        out_shape=jax.ShapeDtypeStruct((2, r, FH), jnp.float32),
    )(x, w2, b2)


def _mm_stats_body(h_ref, a_ref, eps_ref, w_ref, b_ref, z_ref, st_ref):
    i = pl.program_id(0)
    e1 = 1.0 + eps_ref[0, 0]
    zin0 = e1 * h_ref[0] + a_ref[0]
    zin1 = e1 * h_ref[1] + a_ref[1]
    z = (_dot(zin0, w_ref[0:FH, :]) + _dot(zin1, w_ref[FH:2 * FH, :])
         + b_ref[...])
    z_ref[...] = z

    @pl.when(i == 0)
    def _():
        st_ref[...] = jnp.zeros_like(st_ref)
    st_ref[0, :] += jnp.sum(z, axis=0)
    st_ref[1, :] += jnp.sum(z * z, axis=0)


def _mm_stats(h2, a2, eps, w, b, br):
    fo = w.shape[1]
    grid = N // br
    return pl.pallas_call(
        _mm_stats_body,
        grid=(grid,),
        in_specs=[
            pl.BlockSpec((2, br, FH), lambda i: (0, i, 0)),
            pl.BlockSpec((2, br, FH), lambda i: (0, i, 0)),
            pl.BlockSpec(memory_space=pltpu.SMEM),
            pl.BlockSpec((F, fo), lambda i: (0, 0)),
            pl.BlockSpec((1, fo), lambda i: (0, 0)),
        ],
        out_specs=[
            pl.BlockSpec((br, fo), lambda i: (i, 0)),
            pl.BlockSpec((2, fo), lambda i: (0, 0)),
        ],
        out_shape=[
            jax.ShapeDtypeStruct((N, fo), jnp.float32),
            jax.ShapeDtypeStruct((2, fo), jnp.float32),
        ],
    )(h2, a2, eps.reshape(1, 1), w, b.reshape(1, fo))


def _bn_mm_stats_body(z_ref, st_ref, g_ref, bb_ref, w_ref, b_ref,
                      y_ref, st2_ref):
    i = pl.program_id(0)
    inv_n = 1.0 / N
    mu = st_ref[0:1, :] * inv_n
    var = st_ref[1:2, :] * inv_n - mu * mu
    scale = g_ref[...] * jax.lax.rsqrt(var + 1e-5)
    zn = jnp.maximum((z_ref[...] - mu) * scale + bb_ref[...], 0.0)
    y = _dot(zn, w_ref[...]) + b_ref[...]
    y_ref[...] = y

    @pl.when(i == 0)
    def _():
        st2_ref[...] = jnp.zeros_like(st2_ref)
    st2_ref[0, :] += jnp.sum(y, axis=0)
    st2_ref[1, :] += jnp.sum(y * y, axis=0)


def _bn_mm_stats(z, st, g, bb, w, b, br):
    fi = z.shape[1]
    fo = w.shape[1]
    grid = N // br
    return pl.pallas_call(
        _bn_mm_stats_body,
        grid=(grid,),
        in_specs=[
            pl.BlockSpec((br, fi), lambda i: (i, 0)),
            pl.BlockSpec((2, fi), lambda i: (0, 0)),
            pl.BlockSpec((1, fi), lambda i: (0, 0)),
            pl.BlockSpec((1, fi), lambda i: (0, 0)),
            pl.BlockSpec((fi, fo), lambda i: (0, 0)),
            pl.BlockSpec((1, fo), lambda i: (0, 0)),
        ],
        out_specs=[
            pl.BlockSpec((br, fo), lambda i: (i, 0)),
            pl.BlockSpec((2, fo), lambda i: (0, 0)),
        ],
        out_shape=[
            jax.ShapeDtypeStruct((N, fo), jnp.float32),
            jax.ShapeDtypeStruct((2, fo), jnp.float32),
        ],
    )(z, st, g.reshape(1, fi), bb.reshape(1, fi), w, b.reshape(1, fo))


def _bn_relu2_body(y_ref, st_ref, g_ref, bb_ref, h_ref):
    inv_n = 1.0 / N
    mu = st_ref[0:1, :] * inv_n
    var = st_ref[1:2, :] * inv_n - mu * mu
    scale = g_ref[...] * jax.lax.rsqrt(var + 1e-5)
    full = jnp.maximum((y_ref[...] - mu) * scale + bb_ref[...], 0.0)
    h_ref[0] = full[:, 0:FH]
    h_ref[1] = full[:, FH:2 * FH]


def _bn_relu2(y, st, g, bb, br):
    f = y.shape[1]
    grid = N // br
    return pl.pallas_call(
        _bn_relu2_body,
        grid=(grid,),
        in_specs=[
            pl.BlockSpec((br, f), lambda i: (i, 0)),
            pl.BlockSpec((2, f), lambda i: (0, 0)),
            pl.BlockSpec((1, f), lambda i: (0, 0)),
            pl.BlockSpec((1, f), lambda i: (0, 0)),
        ],
        out_specs=pl.BlockSpec((2, br, FH), lambda i: (0, i, 0)),
        out_shape=jax.ShapeDtypeStruct((2, N, FH), jnp.float32),
    )(y, st, g.reshape(1, f), bb.reshape(1, f))


def _pool_body(h_ref, b_ref, w1_ref, b1_ref, w2_ref, b2_ref, o_ref):
    seg = (jax.lax.broadcasted_iota(jnp.int32, (64, N), 0)
           == b_ref[...]).astype(jnp.float32)
    s0 = _dot(seg, h_ref[0])
    s1 = _dot(seg, h_ref[1])
    counts = jnp.maximum(jnp.sum(seg, axis=1, keepdims=True), 1.0)
    p0 = s0 / counts
    p1 = s1 / counts
    hidden = jnp.maximum(
        _dot(p0, w1_ref[0:FH, :]) + _dot(p1, w1_ref[FH:2 * FH, :])
        + b1_ref[...], 0.0)
    o_ref[...] = _dot(hidden, w2_ref[...]) + b2_ref[...]


def _pool(h2, batch, w1, b1, w2, b2):
    f = w1.shape[0]
    return pl.pallas_call(
        _pool_body,
        out_shape=jax.ShapeDtypeStruct((64, f), jnp.float32),
    )(h2, batch.reshape(1, N), w1, b1.reshape(1, f), w2, b2.reshape(1, f))


def _split_w(w):
    fi = w.shape[0]
    return w.reshape(fi, 2, FH).transpose(1, 0, 2)


def _split_b(b):
    return b.reshape(2, 1, FH)


def kernel(x, edge_index, edge_attr, batch, params):
    npad = E2 - E
    pidx = jnp.arange(npad, dtype=jnp.int32)
    src = jnp.concatenate([edge_index[0], (pidx * 97) % N])
    dst = jnp.concatenate([edge_index[1], N + (pidx % NPADROW)])
    src4 = src.reshape(NS, NWIN, K2)
    dst4 = dst.reshape(NS, NWIN, K2)
    ea = jnp.pad(edge_attr, ((0, npad), (0, 0)))

    h2 = _linear2(x, _split_w(params['node_w']), _split_b(params['node_b']),
                  2000)
    e2 = _linear2(ea, _split_w(params['edge_w']), _split_b(params['edge_b']),
                  8192)
    for lp in params['layers']:
        a2 = _sc_msg(h2, e2, src4, dst4)
        z, st1 = _mm_stats(h2, a2, lp['eps'], lp['w1'], lp['b1'], 2000)
        y, st2 = _bn_mm_stats(z, st1, lp['bn1_g'], lp['bn1_b'],
                              lp['w2'], lp['b2'], 2000)
        h2 = _bn_relu2(y, st2, lp['bn_g'], lp['bn_b'], 2000)
    return _pool(h2, batch, params['rw1'], params['rb1'],
                 params['rw2'], params['rb2'])
